# baseline (speedup 1.0000x reference)
import functools

import jax
import jax.numpy as jnp
from jax import lax
from jax.experimental import pallas as pl
from jax.experimental.pallas import tpu as pltpu
from jax.experimental.pallas import tpu_sc as plsc

_CHUNK = 128


def _sc_distmult(h_half, h_par, r_half, r_par, t_half, t_par, ent2, rel2):
    B = h_half.shape[0]
    W = ent2.shape[1]
    D = W // 2
    info = plsc.get_sparse_core_info()
    NC, NS, L = info.num_cores, info.num_subcores, info.num_lanes
    NW = NC * NS
    assert B % (8 * NW) == 0
    bpw = B // NW
    n_chunks = bpw // _CHUNK
    n_groups = _CHUNK // L

    mesh = plsc.VectorSubcoreMesh(core_axis_name="c", subcore_axis_name="s")

    @functools.partial(
        pl.kernel,
        mesh=mesh,
        compiler_params=pltpu.CompilerParams(needs_layout_passes=False),
        out_type=jax.ShapeDtypeStruct((B,), jnp.float32),
        scratch_types=[
            pltpu.VMEM((bpw,), jnp.int32),
            pltpu.VMEM((bpw,), jnp.int32),
            pltpu.VMEM((bpw,), jnp.int32),
            pltpu.VMEM((bpw,), jnp.int32),
            pltpu.VMEM((bpw,), jnp.int32),
            pltpu.VMEM((bpw,), jnp.int32),
            pltpu.VMEM((2, _CHUNK, W), jnp.float32),
            pltpu.VMEM((2, _CHUNK, W), jnp.float32),
            pltpu.VMEM((2, _CHUNK, W), jnp.float32),
            pltpu.VMEM((bpw,), jnp.float32),
            pltpu.SemaphoreType.DMA,
            pltpu.SemaphoreType.DMA,
        ],
    )
    def k(hh_hbm, hp_hbm, rh_hbm, rp_hbm, th_hbm, tp_hbm, ent_hbm, rel_hbm,
          out_hbm, hh, hp, rh, rp, th, tp, eh, rr, et, oc, sem0, sem1):
        wid = lax.axis_index("s") * NC + lax.axis_index("c")
        base = wid * bpw
        pltpu.sync_copy(hh_hbm.at[pl.ds(base, bpw)], hh)
        pltpu.sync_copy(hp_hbm.at[pl.ds(base, bpw)], hp)
        pltpu.sync_copy(rh_hbm.at[pl.ds(base, bpw)], rh)
        pltpu.sync_copy(rp_hbm.at[pl.ds(base, bpw)], rp)
        pltpu.sync_copy(th_hbm.at[pl.ds(base, bpw)], th)
        pltpu.sync_copy(tp_hbm.at[pl.ds(base, bpw)], tp)

        sems = (sem0, sem1)

        def start_chunk(c, buf):
            sl = pl.ds(c * _CHUNK, _CHUNK)
            cp0 = pltpu.async_copy(ent_hbm.at[hh.at[sl]], eh.at[buf], sems[buf])
            cp1 = pltpu.async_copy(rel_hbm.at[rh.at[sl]], rr.at[buf], sems[buf])
            cp2 = pltpu.async_copy(ent_hbm.at[th.at[sl]], et.at[buf], sems[buf])
            return (cp0, cp1, cp2)

        def wait_chunk(cps):
            for cp in cps:
                cp.wait()

        def compute_chunk(c, buf):
            cbase = c * _CHUNK

            def group_body(g, carry):
                rows = g * L + lax.iota(jnp.int32, L)
                off = cbase + g * L
                hpv = hp[pl.ds(off, L)] * D
                rpv = rp[pl.ds(off, L)] * D
                tpv = tp[pl.ds(off, L)] * D

                def d_body(d, acc):
                    a = plsc.load_gather(eh.at[buf], [rows, hpv + d])
                    b = plsc.load_gather(rr.at[buf], [rows, rpv + d])
                    cc = plsc.load_gather(et.at[buf], [rows, tpv + d])
                    return acc + a * b * cc

                acc = lax.fori_loop(0, D, d_body, jnp.zeros((L,), jnp.float32))
                oc[pl.ds(off, L)] = acc
                return carry

            lax.fori_loop(0, n_groups, group_body, 0)

        cps = start_chunk(0, 0)
        for c in range(n_chunks):
            buf = c % 2
            if c + 1 < n_chunks:
                nxt = start_chunk(c + 1, 1 - buf)
            wait_chunk(cps)
            compute_chunk(c, buf)
            if c + 1 < n_chunks:
                cps = nxt
        pltpu.sync_copy(oc, out_hbm.at[pl.ds(base, bpw)])

    return k(h_half, h_par, r_half, r_par, t_half, t_par, ent2, rel2)


def _mlp_body(f_ref, w1_ref, b1_ref, w2_ref, b2_ref, g_ref, bb_ref,
              ws_ref, bs_ref, o_ref):
    f = f_ref[...]
    h = jnp.dot(f, w1_ref[...], preferred_element_type=jnp.float32) + b1_ref[...]
    h = jnp.maximum(h, 0.0)
    h = jnp.dot(h, w2_ref[...], preferred_element_type=jnp.float32) + b2_ref[...]
    mean = jnp.mean(h, axis=-1, keepdims=True)
    var = jnp.mean((h - mean) ** 2, axis=-1, keepdims=True)
    h = (h - mean) * lax.rsqrt(var + 1e-5) * g_ref[...] + bb_ref[...]
    z = jnp.maximum(h, 0.0)
    o_ref[...] = jnp.dot(z, ws_ref[...], preferred_element_type=jnp.float32) + bs_ref[...]


def _tc_meta(feats, W1, b1, W2, b2, ln_g, ln_b, Ws, bs):
    B, F = feats.shape
    D = W1.shape[1]
    block = 2048
    full = lambda s: pl.BlockSpec(s, lambda i: (0,) * len(s))
    out2 = pl.pallas_call(
        _mlp_body,
        grid=(B // block,),
        in_specs=[
            pl.BlockSpec((block, F), lambda i: (i, 0)),
            full((F, D)), full((D,)), full((D, D)), full((D,)),
            full((D,)), full((D,)), full((D, 1)), full((1,)),
        ],
        out_specs=pl.BlockSpec((block, 1), lambda i: (i, 0)),
        out_shape=jax.ShapeDtypeStruct((B, 1), jnp.float32),
    )(feats, W1, b1, W2, b2, ln_g, ln_b, Ws, bs)
    return out2[:, 0]


def kernel(heads, rels, tails, metapath_feats, entity_emb, relation_emb,
           W1, b1, W2, b2, ln_g, ln_b, Ws, bs):
    heads = heads.astype(jnp.int32)
    rels = rels.astype(jnp.int32)
    tails = tails.astype(jnp.int32)
    ent2 = entity_emb.reshape(entity_emb.shape[0] // 2, 128)
    rel2 = relation_emb.reshape(relation_emb.shape[0] // 2, 128)
    distmult = _sc_distmult(
        heads >> 1, heads & 1, rels >> 1, rels & 1, tails >> 1, tails & 1,
        ent2, rel2)
    meta = _tc_meta(metapath_feats, W1, b1, W2, b2, ln_g, ln_b, Ws, bs)
    return distmult + meta
